# Initial kernel scaffold; baseline (speedup 1.0000x reference)
import jax
import jax.numpy as jnp
from jax.experimental import pallas as pl


def kernel(x, table):
    raise NotImplementedError("write your pallas kernel here")


# speedup vs baseline: 3.4122x; 3.4122x over previous
import jax
import jax.numpy as jnp
from jax.experimental import pallas as pl
from jax.experimental.pallas import tpu as pltpu

_BLOCK_ROWS = 512


def kernel(x, table):
    seq_len = x.shape[1]
    d_model = table.shape[1]
    grid = seq_len // _BLOCK_ROWS

    def body(table_ref, out_ref):
        out_ref[0] = table_ref[...]

    out = pl.pallas_call(
        body,
        grid=(grid,),
        in_specs=[
            pl.BlockSpec((_BLOCK_ROWS, d_model), lambda i: (i, 0)),
        ],
        out_specs=pl.BlockSpec((1, _BLOCK_ROWS, d_model), lambda i: (0, i, 0)),
        out_shape=jax.ShapeDtypeStruct((1, seq_len, d_model), table.dtype),
    )(table)
    return out
